# baseline (speedup 1.0000x reference)
import functools

import jax
import jax.numpy as jnp
from jax import lax
from jax.experimental import pallas as pl
from jax.experimental.pallas import tpu as pltpu
from jax.experimental.pallas import tpu_sc as plsc

_BATCH = 16384
_DIM = 16
_NUM_CORES = 2
_NUM_SUBCORES = 16
_NW = _NUM_CORES * _NUM_SUBCORES
_BPW = _BATCH // _NW

_mesh = plsc.VectorSubcoreMesh(core_axis_name="c", subcore_axis_name="s")


@functools.partial(
    pl.kernel,
    out_type=jax.ShapeDtypeStruct((_BATCH,), jnp.float32),
    mesh=_mesh,
    scratch_types=[
        pltpu.VMEM((_BPW,), jnp.int32),
        pltpu.VMEM((_BPW,), jnp.int32),
        pltpu.VMEM((_BPW, _DIM), jnp.float32),
        pltpu.VMEM((_BPW, _DIM), jnp.float32),
        pltpu.VMEM((_BPW,), jnp.float32),
        pltpu.SemaphoreType.DMA,
    ],
    compiler_params=pltpu.CompilerParams(
        needs_layout_passes=False, use_tc_tiling_on_sc=False
    ),
)
def _mf_kernel(s_idx_hbm, u_idx_hbm, s_tab_hbm, u_tab_hbm, out_hbm,
               s_idx_v, u_idx_v, s_rows_v, u_rows_v, out_v, sem):
    wid = lax.axis_index("s") * _NUM_CORES + lax.axis_index("c")
    base = wid * _BPW
    pltpu.sync_copy(s_idx_hbm.at[pl.ds(base, _BPW)], s_idx_v)
    pltpu.sync_copy(u_idx_hbm.at[pl.ds(base, _BPW)], u_idx_v)
    g1 = pltpu.async_copy(s_tab_hbm.at[s_idx_v], s_rows_v, sem)
    g2 = pltpu.async_copy(u_tab_hbm.at[u_idx_v], u_rows_v, sem)
    g1.wait()
    g2.wait()

    lane = lax.iota(jnp.int32, _DIM)
    cols = [(lane + d) & (_DIM - 1) for d in range(_DIM)]

    def group_body(g, carry):
        row = g * _DIM
        row_ids = row + lane
        acc = jnp.zeros((_DIM,), jnp.float32)
        for d in range(_DIM):
            s_col = plsc.load_gather(s_rows_v, [row_ids, cols[d]])
            u_col = plsc.load_gather(u_rows_v, [row_ids, cols[d]])
            acc = acc + s_col * u_col
        out_v[pl.ds(row, _DIM)] = acc
        return carry

    lax.fori_loop(0, _BPW // _DIM, group_body, 0)
    pltpu.sync_copy(out_v, out_hbm.at[pl.ds(base, _BPW)])


def kernel(student_idx, subject_idx, student_table, subject_table):
    return _mf_kernel(student_idx, subject_idx, student_table, subject_table)
